# baseline (speedup 1.0000x reference)
import functools

import jax
import jax.numpy as jnp
from jax import lax
from jax.experimental import pallas as pl
from jax.experimental.pallas import tpu as pltpu
from jax.experimental.pallas import tpu_sc as plsc


def _make_sc_gather(n_groups, row_words, batch):
    info = plsc.get_sparse_core_info()
    nc, ns = info.num_cores, info.num_subcores
    nw = nc * ns
    b_per_w = batch // nw
    ch = 16
    n_ch = b_per_w // ch
    assert batch % (8 * nw) == 0 and b_per_w % ch == 0

    mesh = plsc.VectorSubcoreMesh(core_axis_name="c", subcore_axis_name="s")

    @functools.partial(
        pl.kernel,
        mesh=mesh,
        compiler_params=pltpu.CompilerParams(
            use_tc_tiling_on_sc=False, needs_layout_passes=False
        ),
        out_type=jax.ShapeDtypeStruct((batch, row_words), jnp.float32),
        scratch_types=[
            pltpu.VMEM((b_per_w,), jnp.int32),
            pltpu.VMEM((ch,), jnp.int32),
            pltpu.VMEM((ch, 8, row_words), jnp.float32),
            pltpu.VMEM((ch, row_words), jnp.float32),
            pltpu.SemaphoreType.DMA,
        ],
    )
    def gather_rows(table_hbm, idx_hbm, out_hbm, idx_v, gidx_v, grp_v, rows_v, sem):
        wid = lax.axis_index("s") * nc + lax.axis_index("c")
        base = wid * b_per_w
        pltpu.sync_copy(idx_hbm.at[pl.ds(base, b_per_w)], idx_v)
        lane = lax.broadcasted_iota(jnp.int32, (ch,), 0)
        for c in range(n_ch):
            idx_c = idx_v[pl.ds(c * ch, ch)]
            gidx_v[...] = lax.shift_right_logical(idx_c, 3)
            sub_c = lax.bitwise_and(idx_c, 7)
            pltpu.async_copy(table_hbm.at[gidx_v], grp_v, sem).wait()

            def extract(w, carry, sub_c=sub_c):
                wv = jnp.full((ch,), w, jnp.int32)
                val = plsc.load_gather(grp_v, [lane, sub_c, wv])
                plsc.store_scatter(rows_v, [lane, wv], val)
                return carry

            lax.fori_loop(0, row_words, extract, 0)
            pltpu.sync_copy(rows_v, out_hbm.at[pl.ds(base + c * ch, ch)])

    return gather_rows


def _proj_body(ut_ref, a_ref, o_ref, *, k, o_dim, dk):
    ut = ut_ref[:]
    utb = jnp.concatenate([ut] * k, axis=1)
    rows = lax.broadcasted_iota(jnp.int32, (dk, k * o_dim), 0)
    cols = lax.broadcasted_iota(jnp.int32, (dk, k * o_dim), 1)
    w = jnp.where((cols // o_dim) == (rows % k), utb, 0.0)
    x = jnp.dot(a_ref[:], w, preferred_element_type=jnp.float32)
    for j in range(k):
        o_ref[j] = x[:, j * o_dim:(j + 1) * o_dim]


def kernel(C, V, U, ii):
    n, d, k = V.shape
    _, o_dim, _ = U.shape
    b = ii.shape[0]
    dk = d * k

    v_grp = V.reshape(n // 8, 8, dk)
    idx = ii.astype(jnp.int32)
    gathered = _make_sc_gather(n // 8, dk, b)(v_grp, idx)

    ut = U.transpose(2, 0, 1).reshape(dk, o_dim)

    b_blk = 512
    out = pl.pallas_call(
        functools.partial(_proj_body, k=k, o_dim=o_dim, dk=dk),
        grid=(b // b_blk,),
        in_specs=[
            pl.BlockSpec((dk, o_dim), lambda i: (0, 0)),
            pl.BlockSpec((b_blk, dk), lambda i: (i, 0)),
        ],
        out_specs=pl.BlockSpec((k, b_blk, o_dim), lambda i: (0, i, 0)),
        out_shape=jax.ShapeDtypeStruct((k, b, o_dim), jnp.float32),
    )(ut, gathered)
    return out


# speedup vs baseline: 1.0286x; 1.0286x over previous
import functools

import jax
import jax.numpy as jnp
from jax import lax
from jax.experimental import pallas as pl
from jax.experimental.pallas import tpu as pltpu
from jax.experimental.pallas import tpu_sc as plsc


def _make_sc_gather(n_groups, row_words, batch):
    info = plsc.get_sparse_core_info()
    nc, ns = info.num_cores, info.num_subcores
    nw = nc * ns
    b_per_w = batch // nw
    ch = 16
    n_ch = b_per_w // ch
    assert batch % (8 * nw) == 0 and b_per_w % ch == 0

    mesh = plsc.VectorSubcoreMesh(core_axis_name="c", subcore_axis_name="s")

    @functools.partial(
        pl.kernel,
        mesh=mesh,
        compiler_params=pltpu.CompilerParams(
            use_tc_tiling_on_sc=False, needs_layout_passes=False
        ),
        out_type=jax.ShapeDtypeStruct((batch, row_words), jnp.float32),
        scratch_types=[
            pltpu.VMEM((b_per_w,), jnp.int32),
            pltpu.VMEM((ch,), jnp.int32),
            pltpu.VMEM((ch, 8, row_words), jnp.float32),
            pltpu.VMEM((ch, row_words), jnp.float32),
            pltpu.SemaphoreType.DMA,
        ],
    )
    def gather_rows(table_hbm, idx_hbm, out_hbm, idx_v, gidx_v, grp_v, rows_v, sem):
        wid = lax.axis_index("s") * nc + lax.axis_index("c")
        base = wid * b_per_w
        pltpu.sync_copy(idx_hbm.at[pl.ds(base, b_per_w)], idx_v)
        lane = lax.broadcasted_iota(jnp.int32, (ch,), 0)
        for c in range(n_ch):
            idx_c = idx_v[pl.ds(c * ch, ch)]
            gidx_v[...] = lax.shift_right_logical(idx_c, 3)
            sub_c = lax.bitwise_and(idx_c, 7)
            pltpu.async_copy(table_hbm.at[gidx_v], grp_v, sem).wait()

            @plsc.parallel_loop(0, row_words, unroll=8)
            def extract(w, sub_c=sub_c):
                wv = jnp.full((ch,), w, jnp.int32)
                val = plsc.load_gather(grp_v, [lane, sub_c, wv])
                plsc.store_scatter(rows_v, [lane, wv], val)
            pltpu.sync_copy(rows_v, out_hbm.at[pl.ds(base + c * ch, ch)])

    return gather_rows


def _proj_body(ut_ref, a_ref, o_ref, *, k, o_dim, dk):
    ut = ut_ref[:]
    utb = jnp.concatenate([ut] * k, axis=1)
    rows = lax.broadcasted_iota(jnp.int32, (dk, k * o_dim), 0)
    cols = lax.broadcasted_iota(jnp.int32, (dk, k * o_dim), 1)
    w = jnp.where((cols // o_dim) == (rows % k), utb, 0.0)
    x = jnp.dot(a_ref[:], w, preferred_element_type=jnp.float32)
    for j in range(k):
        o_ref[j] = x[:, j * o_dim:(j + 1) * o_dim]


def kernel(C, V, U, ii):
    n, d, k = V.shape
    _, o_dim, _ = U.shape
    b = ii.shape[0]
    dk = d * k

    v_grp = V.reshape(n // 8, 8, dk)
    idx = ii.astype(jnp.int32)
    gathered = _make_sc_gather(n // 8, dk, b)(v_grp, idx)

    ut = U.transpose(2, 0, 1).reshape(dk, o_dim)

    b_blk = 512
    out = pl.pallas_call(
        functools.partial(_proj_body, k=k, o_dim=o_dim, dk=dk),
        grid=(b // b_blk,),
        in_specs=[
            pl.BlockSpec((dk, o_dim), lambda i: (0, 0)),
            pl.BlockSpec((b_blk, dk), lambda i: (i, 0)),
        ],
        out_specs=pl.BlockSpec((k, b_blk, o_dim), lambda i: (0, i, 0)),
        out_shape=jax.ShapeDtypeStruct((k, b, o_dim), jnp.float32),
    )(ut, gathered)
    return out


# speedup vs baseline: 5.9555x; 5.7900x over previous
import functools

import jax
import jax.numpy as jnp
from jax import lax
from jax.experimental import pallas as pl
from jax.experimental.pallas import tpu as pltpu
from jax.experimental.pallas import tpu_sc as plsc


def _make_sc_gather(row_words, batch):
    info = plsc.get_sparse_core_info()
    nc, ns = info.num_cores, info.num_subcores
    nw = nc * ns
    b_per_w = batch // nw
    assert batch % (8 * nw) == 0

    mesh = plsc.VectorSubcoreMesh(core_axis_name="c", subcore_axis_name="s")

    @functools.partial(
        pl.kernel,
        mesh=mesh,
        out_type=jax.ShapeDtypeStruct((batch, row_words), jnp.float32),
        scratch_types=[
            pltpu.VMEM((b_per_w,), jnp.int32),
            pltpu.VMEM((b_per_w, row_words), jnp.float32),
            pltpu.SemaphoreType.DMA,
        ],
    )
    def gather_rows(table_hbm, idx_hbm, out_hbm, idx_v, rows_v, sem):
        wid = lax.axis_index("s") * nc + lax.axis_index("c")
        base = wid * b_per_w
        pltpu.sync_copy(idx_hbm.at[pl.ds(base, b_per_w)], idx_v)
        pltpu.async_copy(table_hbm.at[idx_v], rows_v, sem).wait()
        pltpu.sync_copy(rows_v, out_hbm.at[pl.ds(base, b_per_w)])

    return gather_rows


def _proj_body(ut_ref, a_ref, o_ref, *, k, o_dim, dk):
    ut = ut_ref[:]
    utb = jnp.concatenate([ut] * k, axis=1)
    rows = lax.broadcasted_iota(jnp.int32, (dk, k * o_dim), 0)
    cols = lax.broadcasted_iota(jnp.int32, (dk, k * o_dim), 1)
    w = jnp.where((cols // o_dim) == (rows % k), utb, 0.0)
    x = jnp.dot(a_ref[:], w, preferred_element_type=jnp.float32)
    for j in range(k):
        o_ref[j] = x[:, j * o_dim:(j + 1) * o_dim]


def kernel(C, V, U, ii):
    n, d, k = V.shape
    _, o_dim, _ = U.shape
    b = ii.shape[0]
    dk = d * k

    v_flat = V.reshape(n, dk)
    idx = ii.astype(jnp.int32)
    gathered = _make_sc_gather(dk, b)(v_flat, idx)

    ut = U.transpose(2, 0, 1).reshape(dk, o_dim)

    b_blk = 512
    out = pl.pallas_call(
        functools.partial(_proj_body, k=k, o_dim=o_dim, dk=dk),
        grid=(b // b_blk,),
        in_specs=[
            pl.BlockSpec((dk, o_dim), lambda i: (0, 0)),
            pl.BlockSpec((b_blk, dk), lambda i: (i, 0)),
        ],
        out_specs=pl.BlockSpec((k, b_blk, o_dim), lambda i: (0, i, 0)),
        out_shape=jax.ShapeDtypeStruct((k, b, o_dim), jnp.float32),
    )(ut, gathered)
    return out
